# baseline (speedup 1.0000x reference)
import functools

import jax
import jax.numpy as jnp
from jax import lax
from jax.experimental import pallas as pl
from jax.experimental.pallas import tpu as pltpu
from jax.experimental.pallas import tpu_sc as plsc

NC = 2
NS = 16
LANE = 16
BC = 128


def _signal_tc_t(yT, W, b):
    S, B = yT.shape
    D = W.shape[0]
    blk = 2048

    def body(yt_ref, w_ref, b_ref, o_ref):
        o_ref[...] = (
            lax.dot_general(
                w_ref[...], yt_ref[...],
                (((1,), (0,)), ((), ())),
                preferred_element_type=jnp.float32,
            )
            + b_ref[...]
        )

    return pl.pallas_call(
        body,
        grid=(B // blk,),
        in_specs=[
            pl.BlockSpec((S, blk), lambda i: (0, i)),
            pl.BlockSpec((D, S), lambda i: (0, 0)),
            pl.BlockSpec((D, 1), lambda i: (0, 0)),
        ],
        out_specs=pl.BlockSpec((D, blk), lambda i: (0, i)),
        out_shape=jax.ShapeDtypeStruct((D, B), jnp.float32),
    )(yT, W, b.reshape(D, 1))


def _make_sc_kernel(B, L, D, nbuf):
    nbt = B // BC
    bc_per_w = nbt // (NC * NS)
    DT = D // 8
    mesh = plsc.VectorSubcoreMesh(core_axis_name="c", subcore_axis_name="s")

    @functools.partial(
        pl.kernel,
        mesh=mesh,
        out_type=jax.ShapeDtypeStruct((L, DT, nbt, 8, BC), jnp.float32),
        scratch_types=[
            pltpu.VMEM((nbuf, BC), jnp.int32),
            pltpu.VMEM((nbuf, BC, D), jnp.float32),
            pltpu.VMEM((nbuf, DT, 8, BC), jnp.float32),
            pltpu.VMEM((DT, 8, BC), jnp.float32),
            pltpu.VMEM_SHARED((NS, DT, 8, BC), jnp.float32),
            [pltpu.SemaphoreType.DMA] * nbuf,
            [pltpu.SemaphoreType.DMA] * nbuf,
            [pltpu.SemaphoreType.DMA] * nbuf,
        ],
        compiler_params=pltpu.CompilerParams(
            use_tc_tiling_on_sc=False, needs_layout_passes=False),
    )
    def sc_k(x_hbm, sig_hbm, emb_hbm, out_hbm,
             idx_v, rows_v, tile_v, sig_v, sig_sh, gsems, wsems, psems):
        sid = lax.axis_index("s")
        wid = sid * NC + lax.axis_index("c")
        ia = lax.iota(jnp.int32, LANE)
        zero_vec = ia - ia
        ds_vec = lax.bitwise_and(ia, 7)
        dt_vecs = [lax.shift_right_logical(ia, 3) + 2 * k
                   for k in range(D // LANE)]

        def unit_start(bc, l, b):
            pltpu.sync_copy(x_hbm.at[l, pl.ds(bc * BC, BC)], idx_v.at[b])
            pltpu.async_copy(emb_hbm.at[idx_v.at[b]], rows_v.at[b], gsems[b])
            pltpu.async_copy(sig_sh.at[sid], tile_v.at[b], psems[b])

        def unit_wait(b):
            pltpu.make_async_copy(
                emb_hbm.at[idx_v.at[b]], rows_v.at[b], gsems[b]).wait()
            pltpu.make_async_copy(
                sig_sh.at[sid], tile_v.at[b], psems[b]).wait()

        def write_start(bc, l, b):
            pltpu.async_copy(
                tile_v.at[b], out_hbm.at[l, :, bc, :, :], wsems[b])

        def write_wait(b):
            pltpu.make_async_copy(
                tile_v.at[b], out_hbm.at[0, :, 0, :, :], wsems[b]).wait()

        def run_chunk(bcl, carry):
            bc = wid * bc_per_w + bcl
            pltpu.sync_copy(sig_hbm.at[:, :, pl.ds(bc * BC, BC)], sig_v)
            pltpu.sync_copy(sig_v, sig_sh.at[sid])
            unit_start(bc, 0, 0)

            def body(h, c1):
                for b in range(nbuf):
                    l = h * nbuf + b
                    nl = l + 1
                    nb = (b + 1) % nbuf

                    @pl.when(nl < L)
                    def _():
                        @pl.when(nl >= nbuf)
                        def _():
                            write_wait(nb)

                        unit_start(bc, nl, nb)

                    unit_wait(b)

                    def bl_body(bl, c2):
                        lane_vec = zero_vec + bl
                        for k in range(D // LANE):
                            vals = rows_v[b, bl, pl.ds(k * LANE, LANE)]
                            plsc.addupdate_scatter(
                                tile_v.at[b],
                                [dt_vecs[k], ds_vec, lane_vec],
                                vals,
                            )
                        return c2

                    lax.fori_loop(0, BC, bl_body, 0, unroll=2)
                    write_start(bc, l, b)
                return c1

            lax.fori_loop(0, L // nbuf, body, 0)
            for b in range(nbuf):
                write_wait(b)
            return carry

        lax.fori_loop(0, bc_per_w, run_chunk, 0)

    return sc_k


def kernel(x, y, embedding, W, b):
    B, L = x.shape
    V, D = embedding.shape
    x2 = (x.T.astype(jnp.int32) * 2)
    emb2 = jnp.pad(embedding, ((0, 0), (0, D))).reshape(2 * V, D)
    sigT = _signal_tc_t(y.T.astype(jnp.float32), W.astype(jnp.float32),
                        b.astype(jnp.float32))
    sig3 = sigT.reshape(D // 8, 8, B)
    sc_k = _make_sc_kernel(B, L, D, nbuf=4)
    out5 = sc_k(x2, sig3, emb2)
    return jnp.transpose(out5, (2, 4, 0, 1, 3)).reshape(B, L, D)


# speedup vs baseline: 3.0013x; 3.0013x over previous
import functools

import jax
import jax.numpy as jnp
from jax import lax
from jax.experimental import pallas as pl
from jax.experimental.pallas import tpu as pltpu
from jax.experimental.pallas import tpu_sc as plsc

NC = 2
NS = 16
LANE = 16


def _signal_tc(y, Wt, b):
    B, S = y.shape
    D = Wt.shape[1]
    blk = 2048

    def body(y_ref, wt_ref, b_ref, o_ref):
        o_ref[...] = (
            lax.dot_general(
                y_ref[...], wt_ref[...],
                (((1,), (0,)), ((), ())),
                preferred_element_type=jnp.float32,
            )
            + b_ref[...]
        )

    return pl.pallas_call(
        body,
        grid=(B // blk,),
        in_specs=[
            pl.BlockSpec((blk, S), lambda i: (i, 0)),
            pl.BlockSpec((S, D), lambda i: (0, 0)),
            pl.BlockSpec((1, D), lambda i: (0, 0)),
        ],
        out_specs=pl.BlockSpec((blk, D), lambda i: (i, 0)),
        out_shape=jax.ShapeDtypeStruct((B, D), jnp.float32),
    )(y, Wt, b.reshape(1, D))


def _make_sc_kernel(B, L, D, bpw, idx_chunk, nbuf):
    n_chunks = L // idx_chunk
    mesh = plsc.VectorSubcoreMesh(core_axis_name="c", subcore_axis_name="s")

    @functools.partial(
        pl.kernel,
        mesh=mesh,
        out_type=jax.ShapeDtypeStruct((B, L, 2 * D), jnp.float32),
        scratch_types=[
            pltpu.VMEM((nbuf, n_chunks, idx_chunk), jnp.int32),
            pltpu.VMEM((nbuf, L, D), jnp.float32),
            pltpu.VMEM((bpw, D), jnp.float32),
            [pltpu.SemaphoreType.DMA] * nbuf,
            [pltpu.SemaphoreType.DMA] * nbuf,
        ],
        compiler_params=pltpu.CompilerParams(use_tc_tiling_on_sc=False),
    )
    def sc_k(x_hbm, sig_hbm, emb_hbm, out_hbm, idx_v, rows_v, sig_v,
             gsems, wsems):
        wid = lax.axis_index("s") * NC + lax.axis_index("c")
        batch0 = wid * bpw
        pltpu.sync_copy(sig_hbm.at[pl.ds(batch0, bpw)], sig_v)

        def gather_start(i, b):
            batch = batch0 + i
            pltpu.sync_copy(
                x_hbm.at[pl.ds(batch * n_chunks, n_chunks)], idx_v.at[b])
            for c in range(n_chunks):
                pltpu.async_copy(
                    emb_hbm.at[idx_v.at[b].at[c]],
                    rows_v.at[b].at[pl.ds(c * idx_chunk, idx_chunk)],
                    gsems[b],
                )

        def gather_wait(b):
            for c in range(n_chunks):
                pltpu.make_async_copy(
                    emb_hbm.at[idx_v.at[b].at[c]],
                    rows_v.at[b].at[pl.ds(c * idx_chunk, idx_chunk)],
                    gsems[b],
                ).wait()

        def write_start(i, b):
            batch = batch0 + i
            pltpu.async_copy(
                rows_v.at[b], out_hbm.at[batch, :, pl.ds(0, D)], wsems[b])

        def write_wait(b):
            pltpu.make_async_copy(
                rows_v.at[b], out_hbm.at[batch0, :, pl.ds(0, D)],
                wsems[b]).wait()

        gather_start(0, 0)

        def body(h, carry):
            for b in range(nbuf):
                i = h * nbuf + b
                j = i + 1
                nb = (b + 1) % nbuf

                @pl.when(j < bpw)
                def _():
                    @pl.when(j >= nbuf)
                    def _():
                        write_wait(nb)

                    gather_start(j, nb)

                gather_wait(b)
                sig = [sig_v[i, pl.ds(k * LANE, LANE)]
                       for k in range(D // LANE)]

                def row_body(r, c2):
                    for k in range(D // LANE):
                        rows_v[b, r, pl.ds(k * LANE, LANE)] += sig[k]
                    return c2

                lax.fori_loop(0, L, row_body, 0, unroll=4)
                write_start(i, b)
            return carry

        lax.fori_loop(0, bpw // nbuf, body, 0)
        for b in range(nbuf):
            write_wait(b)

    return sc_k


def kernel(x, y, embedding, W, b):
    B, L = x.shape
    V, D = embedding.shape
    NW = NC * NS
    bpw = B // NW
    idx_chunk = 100
    x2d = (x.reshape(B * L // idx_chunk, idx_chunk).astype(jnp.int32) * 2)
    emb2 = jnp.pad(embedding, ((0, 0), (0, D))).reshape(2 * V, D)
    Wt = W.T.astype(jnp.float32)
    sig = _signal_tc(y.astype(jnp.float32), Wt, b.astype(jnp.float32))
    sc_k = _make_sc_kernel(B, L, D, bpw, idx_chunk, nbuf=4)
    return sc_k(x2d, sig, emb2)[:, :, :D]
